# baseline (speedup 1.0000x reference)
import functools
import math

import jax
import jax.numpy as jnp
from jax import lax
from jax.experimental import pallas as pl
from jax.experimental.pallas import tpu as pltpu
from jax.experimental.pallas import tpu_sc as plsc

D_MODEL = 128
SCALE = math.sqrt(float(D_MODEL))

NC, NS = 2, 16
NW = NC * NS
CHUNK = 128
SUP = 2
NBUF = 3
LEAD = 2


def _sc_embed(table, idx2d):
    d = table.shape[1]
    chunks_per_w = idx2d.shape[0] // (NW * CHUNK)
    nsup = chunks_per_w // SUP
    rows_sup = SUP * CHUNK
    per_w = chunks_per_w * CHUNK
    num_idx = NW * per_w
    mesh = plsc.VectorSubcoreMesh(core_axis_name="core", subcore_axis_name="subcore")

    scratch = [pltpu.VMEM((per_w,), jnp.int32)]
    scratch += [pltpu.VMEM((rows_sup, d), jnp.float32) for _ in range(NBUF)]
    scratch += [pltpu.SemaphoreType.DMA for _ in range(2 * NBUF)]

    @functools.partial(
        pl.kernel,
        out_type=jax.ShapeDtypeStruct((num_idx, d), table.dtype),
        mesh=mesh,
        scratch_types=scratch,
    )
    def gather_scale(table_hbm, idx_hbm, out_hbm, idx_v, *rest):
        bufs = rest[:NBUF]
        sg = rest[NBUF:2 * NBUF]
        ss = rest[2 * NBUF:]
        wid = lax.axis_index("subcore") * NC + lax.axis_index("core")
        base = wid * per_w

        pltpu.sync_copy(idx_hbm.at[pl.ds(base, per_w)], idx_v)

        def start_gather(j, b):
            pltpu.async_copy(
                table_hbm.at[idx_v.at[pl.ds(j * rows_sup, rows_sup)]], bufs[b], sg[b]
            )

        def wait_gather(j, b):
            pltpu.make_async_copy(
                table_hbm.at[idx_v.at[pl.ds(j * rows_sup, rows_sup)]], bufs[b], sg[b]
            ).wait()

        def start_store(j, b):
            pltpu.async_copy(
                bufs[b], out_hbm.at[pl.ds(base + j * rows_sup, rows_sup)], ss[b]
            )

        def wait_store(b):
            pltpu.make_async_copy(
                bufs[b], out_hbm.at[pl.ds(base, rows_sup)], ss[b]
            ).wait()

        def scale_buf(b):
            @pl.loop(0, rows_sup, step=4)
            def _(r0):
                for dr in range(4):
                    row = bufs[b].at[r0 + dr]
                    for c in range(0, d, 16):
                        row[pl.ds(c, 16)] = row[pl.ds(c, 16)] * SCALE

        def step(j, b):
            g = j + LEAD
            gb = (b + LEAD) % NBUF

            @pl.when(g < nsup)
            def _():
                @pl.when(g >= NBUF)
                def _():
                    wait_store(gb)

                start_gather(g, gb)

            wait_gather(j, b)
            scale_buf(b)
            start_store(j, b)

        for b in range(LEAD):
            start_gather(b, b)

        @pl.loop(0, nsup - 1, step=NBUF)
        def _(j0):
            for i in range(NBUF):
                step(j0 + i, i)

        step(nsup - 1, (nsup - 1) % NBUF)

        for jj in range(nsup - NBUF, nsup):
            wait_store(jj % NBUF)

    return gather_scale(table, idx2d)


def kernel(x, table):
    b, s = x.shape
    idx2d = x.reshape(-1).astype(jnp.int32)
    out = _sc_embed(table, idx2d)
    return out.reshape(b, s, table.shape[1])


# speedup vs baseline: 1.0176x; 1.0176x over previous
import functools
import math

import jax
import jax.numpy as jnp
from jax import lax
from jax.experimental import pallas as pl
from jax.experimental.pallas import tpu as pltpu
from jax.experimental.pallas import tpu_sc as plsc

D_MODEL = 128
SCALE = math.sqrt(float(D_MODEL))

NC, NS = 2, 16
NW = NC * NS
W0 = 128
NBUF = 4
LEAD = 2


def _sc_embed(table, idx):
    n_rows, row_len = idx.shape
    d = table.shape[1]
    rows_per_w = n_rows // NW
    w1 = row_len - W0
    mesh = plsc.VectorSubcoreMesh(core_axis_name="core", subcore_axis_name="subcore")

    scratch = [pltpu.VMEM((rows_per_w, row_len), jnp.int32)]
    scratch += [pltpu.VMEM((row_len, d), jnp.float32) for _ in range(NBUF)]
    scratch += [pltpu.SemaphoreType.DMA for _ in range(2 * NBUF)]

    @functools.partial(
        pl.kernel,
        out_type=jax.ShapeDtypeStruct((n_rows * row_len, d), table.dtype),
        mesh=mesh,
        scratch_types=scratch,
        compiler_params=pltpu.CompilerParams(use_tc_tiling_on_sc=True),
    )
    def gather_scale(table_hbm, idx_hbm, out_hbm, idx_v, *rest):
        bufs = rest[:NBUF]
        sg = rest[NBUF:2 * NBUF]
        ss = rest[2 * NBUF:]
        wid = lax.axis_index("subcore") * NC + lax.axis_index("core")
        row0 = wid * rows_per_w

        pltpu.sync_copy(idx_hbm.at[pl.ds(row0, rows_per_w)], idx_v)

        def start_gather(k, b):
            pltpu.async_copy(
                table_hbm.at[idx_v.at[k, pl.ds(0, W0)]],
                bufs[b].at[pl.ds(0, W0)],
                sg[b],
            )
            pltpu.async_copy(
                table_hbm.at[idx_v.at[k, pl.ds(W0, w1)]],
                bufs[b].at[pl.ds(W0, w1)],
                sg[b],
            )

        def wait_gather(k, b):
            pltpu.make_async_copy(
                table_hbm.at[idx_v.at[k, pl.ds(0, W0)]],
                bufs[b].at[pl.ds(0, W0)],
                sg[b],
            ).wait()
            pltpu.make_async_copy(
                table_hbm.at[idx_v.at[k, pl.ds(W0, w1)]],
                bufs[b].at[pl.ds(W0, w1)],
                sg[b],
            ).wait()

        def start_store(k, b):
            pltpu.async_copy(
                bufs[b],
                out_hbm.at[pl.ds((row0 + k) * row_len, row_len)],
                ss[b],
            )

        def wait_store(b):
            pltpu.make_async_copy(
                bufs[b], out_hbm.at[pl.ds(0, row_len)], ss[b]
            ).wait()

        for b in range(LEAD):
            start_gather(b, b)

        @pl.loop(0, rows_per_w, step=NBUF)
        def _(k0):
            for i in range(NBUF):
                k = k0 + i
                b = i
                g = k + LEAD
                gb = (b + LEAD) % NBUF

                @pl.when(g < rows_per_w)
                def _():
                    @pl.when(g >= NBUF)
                    def _():
                        wait_store(gb)

                    start_gather(g, gb)

                wait_gather(k, b)

                @pl.loop(0, row_len, step=4)
                def _(r0):
                    for dr in range(4):
                        row = bufs[b].at[r0 + dr]
                        for c in range(0, d, 16):
                            row[pl.ds(c, 16)] = row[pl.ds(c, 16)] * SCALE

                start_store(k, b)

        for b in range(NBUF):
            wait_store(b)

    return gather_scale(table, idx)


def kernel(x, table):
    b, s = x.shape
    out = _sc_embed(table, x.astype(jnp.int32))
    return out.reshape(b, s, table.shape[1])
